# baseline (speedup 1.0000x reference)
import dataclasses
import functools

import jax
import jax.numpy as jnp
from jax import lax
from jax.experimental import pallas as pl
from jax.experimental.pallas import tpu as pltpu
from jax.experimental.pallas import tpu_sc as plsc

N_NODES = 10000
N_EDGES = 320000
D = 128
HALF = N_NODES // 2

NC = 2
NS = 16
NT = NC * NS
EPT = N_EDGES // NT
CHUNK = 80
BCH = 16
NCHCAP = 128
SLOTS = NCHCAP * CHUNK
ROWS_PER_SUB = 312


def _matmul(x, W):
    def body(x_ref, w_ref, o_ref):
        o_ref[...] = lax.dot_general(
            x_ref[...], w_ref[...], (((1,), (0,)), ((), ())),
            precision=lax.Precision.HIGHEST,
            preferred_element_type=jnp.float32)

    bm = 2000
    return pl.pallas_call(
        body,
        grid=(N_NODES // bm,),
        in_specs=[
            pl.BlockSpec((bm, D), lambda i: (i, 0)),
            pl.BlockSpec((D, D), lambda i: (0, 0)),
        ],
        out_specs=pl.BlockSpec((bm, D), lambda i: (i, 0)),
        out_shape=jax.ShapeDtypeStruct((N_NODES, D), jnp.float32),
    )(x, W)


_SC_PARAMS = pltpu.CompilerParams()
if "needs_layout_passes" in pltpu.CompilerParams.__dataclass_fields__:
    _SC_PARAMS = dataclasses.replace(_SC_PARAMS, needs_layout_passes=False)

_MESH = plsc.VectorSubcoreMesh(core_axis_name="c", subcore_axis_name="s",
                               num_cores=NC)


@functools.partial(
    pl.kernel,
    mesh=_MESH,
    compiler_params=_SC_PARAMS,
    out_type=[
        jax.ShapeDtypeStruct((NT, 2, SLOTS), jnp.int32),
        jax.ShapeDtypeStruct((NT, 2, SLOTS), jnp.int32),
        jax.ShapeDtypeStruct((NT, 2, SLOTS), jnp.float32),
        jax.ShapeDtypeStruct((NT, 2, 16), jnp.int32),
    ],
    scratch_types=[
        pltpu.VMEM((EPT,), jnp.int32),
        pltpu.VMEM((EPT,), jnp.int32),
        pltpu.VMEM((EPT,), jnp.float32),
        pltpu.VMEM((2, SLOTS), jnp.int32),
        pltpu.VMEM((2, SLOTS), jnp.int32),
        pltpu.VMEM((2, SLOTS), jnp.float32),
        pltpu.VMEM((2, 16), jnp.int32),
        pltpu.SemaphoreType.DMA,
    ],
)
def _sc_partition(cols_hbm, rows_hbm, vals_hbm,
                  pcol_hbm, prow_hbm, pval_hbm, pcnt_hbm,
                  in_c, in_r, in_v, st_c, st_r, st_v, cnt_v, sem):
    cid = lax.axis_index("c")
    sid = lax.axis_index("s")
    wid = sid * NC + cid

    pltpu.async_copy(cols_hbm.at[wid], in_c, sem)
    pltpu.async_copy(rows_hbm.at[wid], in_r, sem)
    pltpu.async_copy(vals_hbm.at[wid], in_v, sem)

    zc = jnp.zeros((16,), jnp.int32)
    zr = jnp.full((16,), HALF, jnp.int32)
    zv = jnp.zeros((16,), jnp.float32)

    @pl.loop(0, SLOTS // 16)
    def _(i):
        for b in range(2):
            st_c[b, pl.ds(i * 16, 16)] = zc
            st_r[b, pl.ds(i * 16, 16)] = zr
            st_v[b, pl.ds(i * 16, 16)] = zv

    pltpu.make_async_copy(cols_hbm.at[wid], in_c, sem).wait()
    pltpu.make_async_copy(rows_hbm.at[wid], in_r, sem).wait()
    pltpu.make_async_copy(vals_hbm.at[wid], in_v, sem).wait()

    b0 = jnp.zeros((16,), jnp.int32)
    b1 = jnp.ones((16,), jnp.int32)

    def body(g, offs):
        o0, o1 = offs
        c16 = in_c[pl.ds(g * 16, 16)]
        r16 = in_r[pl.ds(g * 16, 16)]
        v16 = in_v[pl.ds(g * 16, 16)]
        m0 = r16 < HALF
        m1 = jnp.logical_not(m0)
        i0 = plsc.cumsum(m0.astype(jnp.int32))
        i1 = plsc.cumsum(m1.astype(jnp.int32))
        d0 = o0 + i0 - 1
        d1 = o1 + i1 - 1
        plsc.store_scatter(st_c, [b0, d0], c16, mask=m0)
        plsc.store_scatter(st_r, [b0, d0], r16, mask=m0)
        plsc.store_scatter(st_v, [b0, d0], v16, mask=m0)
        plsc.store_scatter(st_c, [b1, d1], c16, mask=m1)
        plsc.store_scatter(st_r, [b1, d1], r16 - HALF, mask=m1)
        plsc.store_scatter(st_v, [b1, d1], v16, mask=m1)
        n0 = lax.reduce_max(i0, (0,))
        n1 = lax.reduce_max(i1, (0,))
        return (o0 + n0, o1 + n1)

    off0, off1 = lax.fori_loop(0, EPT // 16, body,
                               (jnp.int32(0), jnp.int32(0)))

    blk = BCH * CHUNK
    cnt_v[0, :] = jnp.full((16,), (off0 + blk - 1) // blk, jnp.int32)
    cnt_v[1, :] = jnp.full((16,), (off1 + blk - 1) // blk, jnp.int32)

    pltpu.sync_copy(cnt_v, pcnt_hbm.at[wid])
    pltpu.sync_copy(st_c, pcol_hbm.at[wid])
    pltpu.sync_copy(st_r, prow_hbm.at[wid])
    pltpu.sync_copy(st_v, pval_hbm.at[wid])


@functools.partial(
    pl.kernel,
    mesh=_MESH,
    compiler_params=_SC_PARAMS,
    out_type=jax.ShapeDtypeStruct((N_NODES, D), jnp.float32),
    scratch_types=[
        pltpu.VMEM((BCH, CHUNK), jnp.int32),
        pltpu.VMEM((BCH, CHUNK), jnp.int32),
        pltpu.VMEM((BCH, CHUNK), jnp.float32),
        pltpu.VMEM((CHUNK, D), jnp.float32),
        pltpu.VMEM((CHUNK, D), jnp.float32),
        pltpu.VMEM((CHUNK, D), jnp.float32),
        pltpu.VMEM_SHARED((HALF + 8, D), jnp.float32),
        pltpu.VMEM((16,), jnp.int32),
        pltpu.SemaphoreType.DMA,
        pltpu.SemaphoreType.DMA,
        pltpu.SemaphoreType.DMA,
        pltpu.SemaphoreType.DMA,
        pltpu.SemaphoreType.DMA,
        pltpu.SemaphoreType.DMA,
    ],
)
def _sc_main(xw_hbm, pcol_hbm, prow_hbm, pval_hbm, pcnt_hbm, out_hbm,
             col_v, row_v, val_v, rows_0, rows_1, rows_2, acc_sh, cnt_sm,
             sem_g0, sem_g1, sem_g2, sem_s0, sem_s1, sem_s2):
    cid = lax.axis_index("c")
    sid = lax.axis_index("s")
    bufs = (rows_0, rows_1, rows_2)
    gsems = (sem_g0, sem_g1, sem_g2)
    ssems = (sem_s0, sem_s1, sem_s2)

    @pl.loop(0, 80)
    def _(i):
        for g in range(D // 16):
            rows_0[i, pl.ds(g * 16, 16)] = jnp.zeros((16,), jnp.float32)

    base = pl.multiple_of(sid * ROWS_PER_SUB, 8)
    for k in range(3):
        pltpu.async_copy(rows_0.at[pl.ds(0, 80)],
                         acc_sh.at[pl.ds(base + k * 80, 80)], sem_s0)
    pltpu.async_copy(rows_0.at[pl.ds(0, 72)],
                     acc_sh.at[pl.ds(base + 240, 72)], sem_s0)

    @pl.when(sid == NS - 1)
    def _():
        pltpu.async_copy(rows_0.at[pl.ds(0, 16)],
                         acc_sh.at[pl.ds(NS * ROWS_PER_SUB, 16)], sem_s0)

    for k in range(3):
        pltpu.make_async_copy(rows_0.at[pl.ds(0, 80)],
                              acc_sh.at[pl.ds(base + k * 80, 80)],
                              sem_s0).wait()
    pltpu.make_async_copy(rows_0.at[pl.ds(0, 72)],
                          acc_sh.at[pl.ds(base + 240, 72)], sem_s0).wait()

    @pl.when(sid == NS - 1)
    def _():
        pltpu.make_async_copy(rows_0.at[pl.ds(0, 16)],
                              acc_sh.at[pl.ds(NS * ROWS_PER_SUB, 16)],
                              sem_s0).wait()

    plsc.subcore_barrier()

    def g_issue(j, buf, sem):
        pltpu.async_copy(xw_hbm.at[col_v.at[j]], buf, sem)

    def g_wait(buf, sem):
        pltpu.make_async_copy(xw_hbm.at[col_v.at[0]], buf, sem).wait()

    def scale(j, buf):
        @plsc.parallel_loop(0, CHUNK, step=16)
        def _(e0):
            val16 = val_v[j, pl.ds(e0, 16)]
            for l in range(16):
                bval = val16[l]
                for g in range(D // 16):
                    sl = (e0 + l, pl.ds(g * 16, 16))
                    buf[sl] = buf[sl] * bval

    def s_issue(j, buf, sem):
        pltpu.async_copy(buf, acc_sh.at[row_v.at[j]], sem, add=True)

    def s_wait(buf, sem):
        pltpu.make_async_copy(buf, acc_sh.at[row_v.at[0]], sem).wait()

    for r2 in range(2):
        rsrc = sid * 2 + r2
        pltpu.sync_copy(pcnt_hbm.at[rsrc, cid], cnt_sm)
        nbl = cnt_sm[pl.ds(0, 16)][0]

        def blk_body(bk, carry):
            boff = pl.multiple_of(bk * BCH, 8)
            pltpu.sync_copy(pcol_hbm.at[rsrc, cid, pl.ds(boff, BCH)], col_v)
            pltpu.sync_copy(prow_hbm.at[rsrc, cid, pl.ds(boff, BCH)], row_v)
            pltpu.sync_copy(pval_hbm.at[rsrc, cid, pl.ds(boff, BCH)], val_v)

            g_issue(0, bufs[0], gsems[0])
            g_issue(1, bufs[1], gsems[1])
            g_wait(bufs[0], gsems[0])
            scale(0, bufs[0])
            s_issue(0, bufs[0], ssems[0])
            g_issue(2, bufs[2], gsems[2])

            @pl.loop(0, (BCH - 1) // 3)
            def _(t):
                c0 = 3 * t + 1

                g_wait(bufs[1], gsems[1])
                scale(c0, bufs[1])
                s_issue(c0, bufs[1], ssems[1])
                s_wait(bufs[0], ssems[0])
                g_issue(c0 + 2, bufs[0], gsems[0])

                g_wait(bufs[2], gsems[2])
                scale(c0 + 1, bufs[2])
                s_issue(c0 + 1, bufs[2], ssems[2])
                s_wait(bufs[1], ssems[1])

                @pl.when(c0 + 3 < BCH)
                def _():
                    g_issue(c0 + 3, bufs[1], gsems[1])

                g_wait(bufs[0], gsems[0])
                scale(c0 + 2, bufs[0])
                s_issue(c0 + 2, bufs[0], ssems[0])
                s_wait(bufs[2], ssems[2])

                @pl.when(c0 + 4 < BCH)
                def _():
                    g_issue(c0 + 4, bufs[2], gsems[2])

            s_wait(bufs[0], ssems[0])
            return carry

        lax.fori_loop(0, nbl, blk_body, 0)

    plsc.subcore_barrier()

    gbase = pl.multiple_of(cid * HALF + sid * ROWS_PER_SUB, 8)

    def relu_buf(b, n):
        @pl.loop(0, n)
        def _(i):
            for g in range(D // 16):
                sl = (i, pl.ds(g * 16, 16))
                b[sl] = jnp.maximum(b[sl], 0.0)

    rc = [(k * 80, 80) for k in range(3)] + [(240, 72)]

    def rin(k, issue):
        off, n = rc[k]
        cp = (pltpu.async_copy if issue else pltpu.make_async_copy)(
            acc_sh.at[pl.ds(base + off, n)], bufs[k % 2].at[pl.ds(0, n)],
            gsems[k % 2])
        if not issue:
            cp.wait()

    def rout(k, issue):
        off, n = rc[k]
        cp = (pltpu.async_copy if issue else pltpu.make_async_copy)(
            bufs[k % 2].at[pl.ds(0, n)], out_hbm.at[pl.ds(gbase + off, n)],
            ssems[k % 2])
        if not issue:
            cp.wait()

    rin(0, True)
    for k in range(len(rc)):
        rin(k, False)
        if k + 1 < len(rc):
            if k >= 1:
                rout(k - 1, False)
            rin(k + 1, True)
        relu_buf(bufs[k % 2], rc[k][1])
        rout(k, True)
    rout(len(rc) - 2, False)
    rout(len(rc) - 1, False)

    @pl.when(sid == NS - 1)
    def _():
        pltpu.sync_copy(acc_sh.at[pl.ds(NS * ROWS_PER_SUB, 8)],
                        rows_2.at[pl.ds(0, 8)])
        relu_buf(rows_2, 8)
        pltpu.sync_copy(rows_2.at[pl.ds(0, 8)],
                        out_hbm.at[pl.ds(cid * HALF + NS * ROWS_PER_SUB, 8)])


def _debug_partition_jnp(cols, rows, vals):
    def per_tec(c, r, v):
        flag = (r >= HALF).astype(jnp.int32)
        perm = jnp.argsort(flag, stable=True)
        cs, rs, vs = c[perm], r[perm], v[perm]
        cnt0 = EPT - flag.sum()
        idx = jnp.arange(SLOTS)
        v0 = idx < cnt0
        j0 = jnp.minimum(idx, EPT - 1)
        col0 = jnp.where(v0, cs[j0], 0)
        row0 = jnp.where(v0, rs[j0], HALF)
        val0 = jnp.where(v0, vs[j0], 0.0)
        j1 = jnp.minimum(idx + cnt0, EPT - 1)
        v1 = idx < (EPT - cnt0)
        col1 = jnp.where(v1, cs[j1], 0)
        row1 = jnp.where(v1, rs[j1] - HALF, HALF)
        val1 = jnp.where(v1, vs[j1], 0.0)
        blk = BCH * CHUNK
        n0 = (cnt0 + blk - 1) // blk
        n1 = (EPT - cnt0 + blk - 1) // blk
        return (jnp.stack([col0, col1]), jnp.stack([row0, row1]),
                jnp.stack([val0, val1]),
                jnp.stack([jnp.full((16,), n0, jnp.int32),
                           jnp.full((16,), n1, jnp.int32)]))

    return jax.vmap(per_tec)(cols, rows, vals)


def kernel(x, a_indices, a_values, W):
    xw = _matmul(x, W)
    rows = a_indices[0].reshape(NT, EPT)
    cols = a_indices[1].reshape(NT, EPT)
    vals = a_values.reshape(NT, EPT)
    pcol, prow, pval, pcnt = _sc_partition(cols, rows, vals)
    pcol4 = pcol.reshape(NT, 2, NCHCAP, CHUNK)
    prow4 = prow.reshape(NT, 2, NCHCAP, CHUNK)
    pval4 = pval.reshape(NT, 2, NCHCAP, CHUNK)
    return _sc_main(xw, pcol4, prow4, pval4, pcnt)


# speedup vs baseline: 1.7684x; 1.7684x over previous
import dataclasses
import functools

import jax
import jax.numpy as jnp
from jax import lax
from jax.experimental import pallas as pl
from jax.experimental.pallas import tpu as pltpu
from jax.experimental.pallas import tpu_sc as plsc

N_NODES = 10000
N_EDGES = 320000
D = 128

NS = 16
EPW = N_EDGES // NS
CHUNK = 80
BCH = 25
NBLK = EPW // (CHUNK * BCH)
ROWS_PER_SUB = 624


def _matmul(x, W):
    def body(x_ref, w_ref, o_ref):
        o_ref[...] = lax.dot_general(
            x_ref[...], w_ref[...], (((1,), (0,)), ((), ())),
            precision=lax.Precision.HIGHEST,
            preferred_element_type=jnp.float32)

    bm = 2000
    return pl.pallas_call(
        body,
        grid=(N_NODES // bm,),
        in_specs=[
            pl.BlockSpec((bm, D), lambda i: (i, 0)),
            pl.BlockSpec((D, D), lambda i: (0, 0)),
        ],
        out_specs=pl.BlockSpec((bm, D), lambda i: (i, 0)),
        out_shape=jax.ShapeDtypeStruct((N_NODES, D), jnp.float32),
    )(x, W)


_SC_PARAMS = pltpu.CompilerParams()
if "needs_layout_passes" in pltpu.CompilerParams.__dataclass_fields__:
    _SC_PARAMS = dataclasses.replace(_SC_PARAMS, needs_layout_passes=False)


@functools.partial(
    pl.kernel,
    mesh=plsc.VectorSubcoreMesh(core_axis_name="c", subcore_axis_name="s",
                                num_cores=1),
    compiler_params=_SC_PARAMS,
    out_type=jax.ShapeDtypeStruct((N_NODES, D), jnp.float32),
    scratch_types=[
        pltpu.VMEM((BCH, CHUNK), jnp.int32),
        pltpu.VMEM((BCH, CHUNK), jnp.int32),
        pltpu.VMEM((BCH, CHUNK), jnp.float32),
        pltpu.VMEM((CHUNK, D), jnp.float32),
        pltpu.VMEM((CHUNK, D), jnp.float32),
        pltpu.VMEM((CHUNK, D), jnp.float32),
        pltpu.VMEM_SHARED((N_NODES, D), jnp.float32),
        pltpu.SemaphoreType.DMA,
        pltpu.SemaphoreType.DMA,
        pltpu.SemaphoreType.DMA,
        pltpu.SemaphoreType.DMA,
        pltpu.SemaphoreType.DMA,
        pltpu.SemaphoreType.DMA,
    ],
)
def _sc_scatter(xw_hbm, cols_hbm, rows_hbm, vals_hbm, out_hbm,
                col_v, row_v, val_v, rows_0, rows_1, rows_2, acc_sh,
                sem_g0, sem_g1, sem_g2, sem_s0, sem_s1, sem_s2):
    sid = lax.axis_index("s")
    bufs = (rows_0, rows_1, rows_2)
    gsems = (sem_g0, sem_g1, sem_g2)
    ssems = (sem_s0, sem_s1, sem_s2)

    @pl.loop(0, 80)
    def _(i):
        for g in range(D // 16):
            rows_0[i, pl.ds(g * 16, 16)] = jnp.zeros((16,), jnp.float32)

    base = pl.multiple_of(sid * ROWS_PER_SUB, 8)
    for k in range(7):
        pltpu.async_copy(rows_0.at[pl.ds(0, 80)],
                         acc_sh.at[pl.ds(base + k * 80, 80)], sem_s0)
    pltpu.async_copy(rows_0.at[pl.ds(0, 64)],
                     acc_sh.at[pl.ds(base + 560, 64)], sem_s0)

    @pl.when(sid == NS - 1)
    def _():
        pltpu.async_copy(rows_0.at[pl.ds(0, 16)],
                         acc_sh.at[pl.ds(NS * ROWS_PER_SUB, 16)], sem_s0)

    for k in range(7):
        pltpu.make_async_copy(rows_0.at[pl.ds(0, 80)],
                              acc_sh.at[pl.ds(base + k * 80, 80)],
                              sem_s0).wait()
    pltpu.make_async_copy(rows_0.at[pl.ds(0, 64)],
                          acc_sh.at[pl.ds(base + 560, 64)], sem_s0).wait()

    @pl.when(sid == NS - 1)
    def _():
        pltpu.make_async_copy(rows_0.at[pl.ds(0, 16)],
                              acc_sh.at[pl.ds(NS * ROWS_PER_SUB, 16)],
                              sem_s0).wait()

    plsc.subcore_barrier()

    def g_issue(j, buf, sem):
        pltpu.async_copy(xw_hbm.at[col_v.at[j]], buf, sem)

    def g_wait(buf, sem):
        pltpu.make_async_copy(xw_hbm.at[col_v.at[0]], buf, sem).wait()

    def scale(j, buf):
        @plsc.parallel_loop(0, CHUNK, step=16)
        def _(e0):
            val16 = val_v[j, pl.ds(e0, 16)]
            for l in range(16):
                bval = val16[l]
                for g in range(D // 16):
                    sl = (e0 + l, pl.ds(g * 16, 16))
                    buf[sl] = buf[sl] * bval

    def s_issue(j, buf, sem):
        pltpu.async_copy(buf, acc_sh.at[row_v.at[j]], sem, add=True)

    def s_wait(buf, sem):
        pltpu.make_async_copy(buf, acc_sh.at[row_v.at[0]], sem).wait()

    @pl.loop(0, NBLK)
    def _(b):
        pltpu.sync_copy(cols_hbm.at[sid, b], col_v)
        pltpu.sync_copy(rows_hbm.at[sid, b], row_v)
        pltpu.sync_copy(vals_hbm.at[sid, b], val_v)

        g_issue(0, bufs[0], gsems[0])
        g_issue(1, bufs[1], gsems[1])
        g_wait(bufs[0], gsems[0])
        scale(0, bufs[0])
        s_issue(0, bufs[0], ssems[0])
        g_issue(2, bufs[2], gsems[2])

        @pl.loop(0, (BCH - 1) // 3)
        def _(t):
            c0 = 3 * t + 1

            g_wait(bufs[1], gsems[1])
            scale(c0, bufs[1])
            s_issue(c0, bufs[1], ssems[1])
            s_wait(bufs[0], ssems[0])
            g_issue(c0 + 2, bufs[0], gsems[0])

            g_wait(bufs[2], gsems[2])
            scale(c0 + 1, bufs[2])
            s_issue(c0 + 1, bufs[2], ssems[2])
            s_wait(bufs[1], ssems[1])

            @pl.when(c0 + 3 < BCH)
            def _():
                g_issue(c0 + 3, bufs[1], gsems[1])

            g_wait(bufs[0], gsems[0])
            scale(c0 + 2, bufs[0])
            s_issue(c0 + 2, bufs[0], ssems[0])
            s_wait(bufs[2], ssems[2])

            @pl.when(c0 + 4 < BCH)
            def _():
                g_issue(c0 + 4, bufs[2], gsems[2])

        s_wait(bufs[0], ssems[0])

    plsc.subcore_barrier()

    def relu_buf(b, n):
        @pl.loop(0, n)
        def _(i):
            for g in range(D // 16):
                sl = (i, pl.ds(g * 16, 16))
                b[sl] = jnp.maximum(b[sl], 0.0)

    rc = [(base + k * 80, 80) for k in range(7)] + [(base + 560, 64)]

    def rin(k, issue):
        off, n = rc[k]
        cp = (pltpu.async_copy if issue else pltpu.make_async_copy)(
            acc_sh.at[pl.ds(off, n)], bufs[k % 2].at[pl.ds(0, n)],
            gsems[k % 2])
        if not issue:
            cp.wait()

    def rout(k, issue):
        off, n = rc[k]
        cp = (pltpu.async_copy if issue else pltpu.make_async_copy)(
            bufs[k % 2].at[pl.ds(0, n)], out_hbm.at[pl.ds(off, n)],
            ssems[k % 2])
        if not issue:
            cp.wait()

    rin(0, True)
    for k in range(len(rc)):
        rin(k, False)
        if k + 1 < len(rc):
            if k >= 1:
                rout(k - 1, False)
            rin(k + 1, True)
        relu_buf(bufs[k % 2], rc[k][1])
        rout(k, True)
    rout(len(rc) - 2, False)
    rout(len(rc) - 1, False)

    @pl.when(sid == NS - 1)
    def _():
        pltpu.sync_copy(acc_sh.at[pl.ds(NS * ROWS_PER_SUB, 16)],
                        rows_2.at[pl.ds(0, 16)])
        relu_buf(rows_2, 16)
        pltpu.sync_copy(rows_2.at[pl.ds(0, 16)],
                        out_hbm.at[pl.ds(NS * ROWS_PER_SUB, 16)])


def kernel(x, a_indices, a_values, W):
    xw = _matmul(x, W)
    rows = a_indices[0].reshape(NS, NBLK, BCH, CHUNK)
    cols = a_indices[1].reshape(NS, NBLK, BCH, CHUNK)
    vals = a_values.reshape(NS, NBLK, BCH, CHUNK)
    return _sc_scatter(xw, cols, rows, vals)
